# Initial kernel scaffold; baseline (speedup 1.0000x reference)
import jax
import jax.numpy as jnp
from jax.experimental import pallas as pl


def kernel(nodes, neighs, table):
    raise NotImplementedError("write your pallas kernel here")


# speedup vs baseline: 3.9391x; 3.9391x over previous
import functools

import jax
import jax.numpy as jnp
from jax import lax
from jax.experimental import pallas as pl
from jax.experimental.pallas import tpu as pltpu
from jax.experimental.pallas import tpu_sc as plsc

NC = 2
NS = 16
LANES = 16
NW = NC * NS
NBINS = 10240
KPAD = 40


def _wid():
    return lax.axis_index("s") * NC + lax.axis_index("c")


def _sc_mesh():
    return plsc.VectorSubcoreMesh(
        core_axis_name="c", subcore_axis_name="s",
        num_cores=NC, num_subcores=NS)


_SC_PARAMS = pltpu.CompilerParams(needs_layout_passes=False)


def _stats_body(K, RPW, idx_hbm, pf_hbm, ff_hbm, rs_hbm,
                idx_v, f_v, hist_v, rs_v):
    wid = _wid()
    pltpu.sync_copy(idx_hbm.at[wid], idx_v)

    zeros16 = jnp.zeros((LANES,), jnp.float32)

    @pl.loop(0, NBINS // LANES)
    def _zero(i):
        hist_v[pl.ds(i * LANES, LANES)] = zeros16

    @pl.loop(0, RPW // LANES)
    def _groups(g):
        base = g * LANES
        v = [idx_v[pl.ds(j * RPW + base, LANES)] for j in range(K)]
        rowcnt = zeros16
        for j in range(K):
            cnt = jnp.zeros((LANES,), jnp.int32)
            for k in range(j):
                cnt = cnt + jnp.where(v[j] == v[k], 1, 0).astype(jnp.int32)
            fj = jnp.where(cnt == 0, 1.0, 0.0).astype(jnp.float32)
            f_v[pl.ds(j * RPW + base, LANES)] = fj
            rowcnt = rowcnt + fj
        rs_v[pl.ds(base, LANES)] = rowcnt

    iota = lax.iota(jnp.int32, LANES)
    ones16 = jnp.ones((LANES,), jnp.float32)

    @pl.loop(0, RPW)
    def _histrows(r):
        rsplat = jnp.broadcast_to(jnp.int32(0) + r, (LANES,))
        for c in range(3):
            jv = iota + c * LANES
            jc = jnp.minimum(jv, K - 1)
            flat = jc * RPW + rsplat
            vals = plsc.load_gather(idx_v, [flat])
            fv = plsc.load_gather(f_v, [flat])
            m = jnp.logical_and(jv < K, fv > 0.5)
            plsc.addupdate_scatter(hist_v, [vals], ones16, mask=m)

    pltpu.sync_copy(f_v, ff_hbm.at[wid])
    pltpu.sync_copy(rs_v, rs_hbm.at[wid])
    pltpu.sync_copy(hist_v, pf_hbm.at[wid])


def _norm_body(pf_ref, rs_ref, cs_ref, rsc_ref):
    cs = jnp.sum(pf_ref[...], axis=0, keepdims=True)
    cs_ref[...] = jnp.where(cs > 0.0, lax.rsqrt(cs), 1.0)
    rsc_ref[...] = lax.rsqrt(rs_ref[...])


def _gather_body(K, RPW, D, idx_hbm, ff_hbm, cs_hbm, rsc_hbm, tab_hbm,
                 out_hbm, idx_v, f_v, cs_v, rs_v, coef_v, lists_v, rows_v,
                 oblk_v, sem):
    NCH = D // LANES
    wid = _wid()
    pltpu.sync_copy(idx_hbm.at[wid], idx_v)
    pltpu.sync_copy(ff_hbm.at[wid], f_v)
    pltpu.sync_copy(cs_hbm, cs_v)
    pltpu.sync_copy(rsc_hbm.at[wid], rs_v)

    @pl.loop(0, RPW // LANES)
    def _coef(g):
        base = g * LANES
        rsv = rs_v[pl.ds(base, LANES)]
        for j in range(K):
            vj = idx_v[pl.ds(j * RPW + base, LANES)]
            csg = plsc.load_gather(cs_v, [vj])
            coef_v[pl.ds(j * RPW + base, LANES)] = (
                f_v[pl.ds(j * RPW + base, LANES)] * csg * rsv)

    iota = lax.iota(jnp.int32, LANES)

    @pl.loop(0, RPW)
    def _lists(r):
        rsplat = jnp.broadcast_to(jnp.int32(0) + r, (LANES,))
        for c in range(2):
            jv = iota + c * LANES
            vals = plsc.load_gather(idx_v, [jv * RPW + rsplat])
            lists_v[pl.ds(r * KPAD + c * LANES, LANES)] = vals
        jv = iota + 2 * LANES
        jc = jnp.minimum(jv, K - 1)
        vals = plsc.load_gather(idx_v, [jc * RPW + rsplat])
        col = jnp.minimum(jv, KPAD - 1)
        plsc.store_scatter(lists_v, [rsplat * KPAD + col], vals,
                           mask=jv < KPAD)

    @pl.loop(0, RPW)
    def _rows(r):
        pltpu.async_copy(
            tab_hbm.at[lists_v.at[pl.ds(r * KPAD, KPAD)]], rows_v, sem
        ).wait()
        rsplat = jnp.broadcast_to(jnp.int32(0) + r, (LANES,))
        cvals = []
        for c in range((K + LANES - 1) // LANES):
            jv = iota + c * LANES
            jc = jnp.minimum(jv, K - 1)
            cvals.append(plsc.load_gather(coef_v, [jc * RPW + rsplat]))
        acc = [jnp.zeros((LANES,), jnp.float32) for _ in range(NCH)]
        for j in range(K):
            s = cvals[j // LANES][j % LANES]
            for c in range(NCH):
                acc[c] = acc[c] + s * rows_v[j, pl.ds(c * LANES, LANES)]
        for c in range(NCH):
            oblk_v[r, pl.ds(c * LANES, LANES)] = acc[c]

    pltpu.sync_copy(oblk_v, out_hbm.at[pl.ds(wid * RPW, RPW)])


@functools.lru_cache(maxsize=None)
def _build(B, K, N, D):
    RPW = B // NW
    f32 = jnp.float32

    stats = pl.kernel(
        functools.partial(_stats_body, K, RPW),
        out_type=[
            jax.ShapeDtypeStruct((NW, NBINS), f32),
            jax.ShapeDtypeStruct((NW, K * RPW), f32),
            jax.ShapeDtypeStruct((NW, RPW), f32),
        ],
        mesh=_sc_mesh(),
        compiler_params=_SC_PARAMS,
        scratch_types=[
            pltpu.VMEM((K * RPW,), jnp.int32),
            pltpu.VMEM((K * RPW,), f32),
            pltpu.VMEM((NBINS,), f32),
            pltpu.VMEM((RPW,), f32),
        ],
    )

    norm = pl.pallas_call(
        _norm_body,
        out_shape=[
            jax.ShapeDtypeStruct((1, NBINS), f32),
            jax.ShapeDtypeStruct((NW, RPW), f32),
        ],
    )

    gather = pl.kernel(
        functools.partial(_gather_body, K, RPW, D),
        out_type=jax.ShapeDtypeStruct((B, D), f32),
        mesh=_sc_mesh(),
        compiler_params=_SC_PARAMS,
        scratch_types=[
            pltpu.VMEM((K * RPW,), jnp.int32),
            pltpu.VMEM((K * RPW,), f32),
            pltpu.VMEM((NBINS,), f32),
            pltpu.VMEM((RPW,), f32),
            pltpu.VMEM((K * RPW,), f32),
            pltpu.VMEM((RPW * KPAD,), jnp.int32),
            pltpu.VMEM((KPAD, D), f32),
            pltpu.VMEM((RPW, D), f32),
            pltpu.SemaphoreType.DMA,
        ],
    )
    return stats, norm, gather


def kernel(nodes, neighs, table):
    B, DEG = neighs.shape
    K = DEG + 1
    N, D = table.shape
    RPW = B // NW
    stats, norm, gather = _build(B, K, N, D)

    all_idx = jnp.concatenate([neighs, nodes[:, None]], axis=1)
    idx_blocks = all_idx.reshape(NW, RPW, K).transpose(0, 2, 1).reshape(
        NW, K * RPW)

    partials, fflags, rowsum = stats(idx_blocks)
    colscale2, rowscale = norm(partials, rowsum)
    colscale = colscale2.reshape(NBINS)
    out = gather(idx_blocks, fflags, colscale, rowscale, table)
    return out


# speedup vs baseline: 8.2170x; 2.0860x over previous
import functools

import jax
import jax.numpy as jnp
from jax import lax
from jax.experimental import pallas as pl
from jax.experimental.pallas import tpu as pltpu
from jax.experimental.pallas import tpu_sc as plsc

NC = 2
NS = 16
LANES = 16
NW = NC * NS
NBINS = 10240
KPAD = 40


def _wid():
    return lax.axis_index("s") * NC + lax.axis_index("c")


def _sc_mesh():
    return plsc.VectorSubcoreMesh(
        core_axis_name="c", subcore_axis_name="s",
        num_cores=NC, num_subcores=NS)


_SC_PARAMS = pltpu.CompilerParams(needs_layout_passes=False)


def _stats_body(K, RPW, idx_hbm, pf_hbm, ff_hbm, rs_hbm,
                idx_v, f_v, hist_v, rs_v):
    wid = _wid()
    pltpu.sync_copy(idx_hbm.at[wid], idx_v)

    zeros16 = jnp.zeros((LANES,), jnp.float32)

    @pl.loop(0, NBINS // LANES)
    def _zero(i):
        hist_v[pl.ds(i * LANES, LANES)] = zeros16

    @pl.loop(0, RPW // LANES)
    def _groups(g):
        base = g * LANES
        v = [idx_v[pl.ds(j * RPW + base, LANES)] for j in range(K)]
        rowcnt = zeros16
        for j in range(K):
            cnt = jnp.zeros((LANES,), jnp.int32)
            for k in range(j):
                cnt = cnt + jnp.where(v[j] == v[k], 1, 0).astype(jnp.int32)
            fj = jnp.where(cnt == 0, 1.0, 0.0).astype(jnp.float32)
            f_v[pl.ds(j * RPW + base, LANES)] = fj
            rowcnt = rowcnt + fj
        rs_v[pl.ds(base, LANES)] = rowcnt

    iota = lax.iota(jnp.int32, LANES)
    ones16 = jnp.ones((LANES,), jnp.float32)

    @pl.loop(0, RPW)
    def _histrows(r):
        rsplat = jnp.broadcast_to(jnp.int32(0) + r, (LANES,))
        for c in range(3):
            jv = iota + c * LANES
            jc = jnp.minimum(jv, K - 1)
            flat = jc * RPW + rsplat
            vals = plsc.load_gather(idx_v, [flat])
            fv = plsc.load_gather(f_v, [flat])
            m = jnp.logical_and(jv < K, fv > 0.5)
            plsc.addupdate_scatter(hist_v, [vals], ones16, mask=m)

    pltpu.sync_copy(f_v, ff_hbm.at[wid])
    pltpu.sync_copy(rs_v, rs_hbm.at[wid])
    pltpu.sync_copy(hist_v, pf_hbm.at[wid])


def _norm_body(pf_ref, rs_ref, cs_ref, rsc_ref):
    cs = jnp.sum(pf_ref[...], axis=0, keepdims=True)
    cs_ref[...] = jnp.where(cs > 0.0, lax.rsqrt(cs), 1.0)
    rsc_ref[...] = lax.rsqrt(rs_ref[...])


def _gather_body(K, RPW, D, idx_hbm, ff_hbm, cs_hbm, rsc_hbm, tab_hbm,
                 out_hbm, idx_v, f_v, cs_v, rs_v, coef_v, lists_v, rows_v,
                 oblk_v, sem0, sem1):
    NCH = D // LANES
    RB = 8
    wid = _wid()
    pltpu.sync_copy(idx_hbm.at[wid], idx_v)
    pltpu.sync_copy(ff_hbm.at[wid], f_v)
    pltpu.sync_copy(cs_hbm, cs_v)
    pltpu.sync_copy(rsc_hbm.at[wid], rs_v)

    iota = lax.iota(jnp.int32, LANES)

    @pl.loop(0, RPW)
    def _lists(r):
        rsplat = jnp.broadcast_to(jnp.int32(0) + r, (LANES,))
        for c in range((K + LANES - 1) // LANES):
            jv = iota + c * LANES
            jc = jnp.minimum(jv, K - 1)
            vals = plsc.load_gather(idx_v, [jc * RPW + rsplat])
            plsc.store_scatter(lists_v, [rsplat * K + jv], vals,
                               mask=jv < K)

    BLK = K * RB
    NBLK = RPW // RB
    sems = (sem0, sem1)

    def _fire(q, b):
        return pltpu.async_copy(
            tab_hbm.at[lists_v.at[pl.ds(q * BLK, BLK)]],
            rows_v.at[b], sems[b])

    _fire(0, 0)
    _fire(1, 1)

    @pl.loop(0, RPW // LANES)
    def _coef(g):
        base = g * LANES
        rsv = rs_v[pl.ds(base, LANES)]
        for j in range(K):
            vj = idx_v[pl.ds(j * RPW + base, LANES)]
            csg = plsc.load_gather(cs_v, [vj])
            coef_v[pl.ds(j * RPW + base, LANES)] = (
                f_v[pl.ds(j * RPW + base, LANES)] * csg * rsv)

    @pl.loop(0, NBLK, step=2)
    def _blocks(q0):
        for b in range(2):
            q = q0 + b
            pltpu.make_async_copy(
                tab_hbm.at[lists_v.at[pl.ds(q * BLK, BLK)]],
                rows_v.at[b], sems[b]).wait()

            @pl.loop(0, RB)
            def _rowloop(rb):
                r = q * RB + rb
                rsplat = jnp.broadcast_to(jnp.int32(0) + r, (LANES,))
                cvals = []
                for c in range((K + LANES - 1) // LANES):
                    jv = iota + c * LANES
                    jc = jnp.minimum(jv, K - 1)
                    cvals.append(
                        plsc.load_gather(coef_v, [jc * RPW + rsplat]))
                acc = [jnp.zeros((LANES,), jnp.float32)
                       for _ in range(NCH)]
                rbK = rb * K
                for j in range(K):
                    s = cvals[j // LANES][j % LANES]
                    for c in range(NCH):
                        acc[c] = acc[c] + s * rows_v[
                            b, rbK + j, pl.ds(c * LANES, LANES)]
                for c in range(NCH):
                    oblk_v[r, pl.ds(c * LANES, LANES)] = acc[c]

            @pl.when(q + 2 < NBLK)
            def _next():
                _fire(q + 2, b)

    pltpu.sync_copy(oblk_v, out_hbm.at[pl.ds(wid * RPW, RPW)])


@functools.lru_cache(maxsize=None)
def _build(B, K, N, D):
    RPW = B // NW
    f32 = jnp.float32

    stats = pl.kernel(
        functools.partial(_stats_body, K, RPW),
        out_type=[
            jax.ShapeDtypeStruct((NW, NBINS), f32),
            jax.ShapeDtypeStruct((NW, K * RPW), f32),
            jax.ShapeDtypeStruct((NW, RPW), f32),
        ],
        mesh=_sc_mesh(),
        compiler_params=_SC_PARAMS,
        scratch_types=[
            pltpu.VMEM((K * RPW,), jnp.int32),
            pltpu.VMEM((K * RPW,), f32),
            pltpu.VMEM((NBINS,), f32),
            pltpu.VMEM((RPW,), f32),
        ],
    )

    norm = pl.pallas_call(
        _norm_body,
        out_shape=[
            jax.ShapeDtypeStruct((1, NBINS), f32),
            jax.ShapeDtypeStruct((NW, RPW), f32),
        ],
    )

    gather = pl.kernel(
        functools.partial(_gather_body, K, RPW, D),
        out_type=jax.ShapeDtypeStruct((B, D), f32),
        mesh=_sc_mesh(),
        compiler_params=_SC_PARAMS,
        scratch_types=[
            pltpu.VMEM((K * RPW,), jnp.int32),
            pltpu.VMEM((K * RPW,), f32),
            pltpu.VMEM((NBINS,), f32),
            pltpu.VMEM((RPW,), f32),
            pltpu.VMEM((K * RPW,), f32),
            pltpu.VMEM((K * RPW,), jnp.int32),
            pltpu.VMEM((2, K * 8, D), f32),
            pltpu.VMEM((RPW, D), f32),
            pltpu.SemaphoreType.DMA,
            pltpu.SemaphoreType.DMA,
        ],
    )
    return stats, norm, gather


def kernel(nodes, neighs, table):
    B, DEG = neighs.shape
    K = DEG + 1
    N, D = table.shape
    RPW = B // NW
    stats, norm, gather = _build(B, K, N, D)

    all_idx = jnp.concatenate([neighs, nodes[:, None]], axis=1)
    idx_blocks = all_idx.reshape(NW, RPW, K).transpose(0, 2, 1).reshape(
        NW, K * RPW)

    partials, fflags, rowsum = stats(idx_blocks)
    colscale2, rowscale = norm(partials, rowsum)
    colscale = colscale2.reshape(NBINS)
    out = gather(idx_blocks, fflags, colscale, rowscale, table)
    return out
